# Initial kernel scaffold; baseline (speedup 1.0000x reference)
import jax
import jax.numpy as jnp
from jax.experimental import pallas as pl


def kernel(features, labels):
    raise NotImplementedError("write your pallas kernel here")


# speedup vs baseline: 7.2539x; 7.2539x over previous
import jax
import jax.numpy as jnp
from jax import lax
from jax.experimental import pallas as pl
from jax.experimental.pallas import tpu as pltpu

_NCLS = 13
_CPAD = 16
_R = 2048
_NBLK = 32


def _body(lab_ref, x_ref, out_ref, acc_f, acc_a):
    i = pl.program_id(0)
    x = x_ref[...]
    lab = lab_ref[0]
    cls = lax.broadcasted_iota(jnp.int32, (_CPAD, _R), 0)
    oh = (cls == lab).astype(jnp.float32)
    rowsq = jnp.sum(x * x, axis=1, keepdims=True)
    colid = lax.broadcasted_iota(jnp.int32, (_R, 128), 1)
    aux = jnp.where(colid == 0, rowsq,
                    jnp.where(colid == 1, 1.0, 0.0))
    pf = lax.dot(oh, x, precision=lax.Precision.HIGHEST,
                 preferred_element_type=jnp.float32)
    pa = lax.dot(oh, aux, precision=lax.Precision.HIGHEST,
                 preferred_element_type=jnp.float32)

    @pl.when(i == 0)
    def _init():
        acc_f[...] = pf
        acc_a[...] = pa

    @pl.when(i > 0)
    def _accum():
        acc_f[...] += pf
        acc_a[...] += pa

    @pl.when(i == pl.num_programs(0) - 1)
    def _finish():
        af = acc_f[...]
        aa = acc_a[...]
        caux = lax.broadcasted_iota(jnp.int32, (_CPAD, 128), 1)
        counts = jnp.sum(jnp.where(caux == 1, aa, 0.0), axis=1, keepdims=True)
        sumsq = jnp.sum(jnp.where(caux == 0, aa, 0.0), axis=1, keepdims=True)
        safe = jnp.maximum(counts, 1.0)
        nrm = jnp.sum(af * af, axis=1, keepdims=True)
        var = (sumsq - nrm / safe) / safe
        rid = lax.broadcasted_iota(jnp.int32, (_CPAD, 1), 0)
        valid = (counts > 1.0) & (rid < _NCLS)
        vc = jnp.sum(jnp.where(valid, 1.0, 0.0), axis=(0, 1), keepdims=True)
        loss = jnp.sum(jnp.where(valid, var, 0.0), axis=(0, 1), keepdims=True)
        loss = jnp.where(vc > 0, loss / jnp.maximum(vc, 1.0), 0.0)
        out_ref[...] = loss


def kernel(features, labels):
    lab3 = labels.reshape(_NBLK, 1, _R)
    out = pl.pallas_call(
        _body,
        grid=(_NBLK,),
        in_specs=[
            pl.BlockSpec((1, 1, _R), lambda i: (i, 0, 0)),
            pl.BlockSpec((_R, 256), lambda i: (i, 0)),
        ],
        out_specs=pl.BlockSpec((1, 1), lambda i: (0, 0)),
        out_shape=jax.ShapeDtypeStruct((1, 1), jnp.float32),
        scratch_shapes=[
            pltpu.VMEM((_CPAD, 256), jnp.float32),
            pltpu.VMEM((_CPAD, 128), jnp.float32),
        ],
    )(lab3, features)
    return out[0, 0]
